# speedup vs baseline: 3.1621x; 3.1621x over previous
import functools

import jax
import jax.numpy as jnp
from jax import lax
from jax.experimental import pallas as pl
from jax.experimental.pallas import tpu as pltpu
from jax.experimental.pallas import tpu_sc as plsc

N = 320000
D = 128
B = 1024
NC = 2
NS = 16
NW = NC * NS
CHUNK = 125
CPW = N // NW // CHUNK
BPS = B // NS


def _sc_segment_sum(rows3, ids3):
  mesh = plsc.VectorSubcoreMesh(core_axis_name="c", subcore_axis_name="s")

  @functools.partial(
      pl.kernel,
      mesh=mesh,
      out_type=jax.ShapeDtypeStruct((NC, B, D), jnp.float32),
      scratch_types=[
          pltpu.VMEM((CPW, CHUNK), jnp.int32),
          pltpu.VMEM((CHUNK, D), jnp.float32),
          pltpu.VMEM((BPS, D), jnp.float32),
          pltpu.VMEM_SHARED((B, D), jnp.float32),
      ],
  )
  def k(rows_hbm, ids_hbm, out_hbm, ids_v, buf_v, tmp_v, acc_sh):
    cid = lax.axis_index("c")
    sid = lax.axis_index("s")
    wid = cid * NS + sid

    def zrow(r, carry):
      for c in range(D // 16):
        tmp_v[r, pl.ds(c * 16, 16)] = jnp.zeros((16,), jnp.float32)
      return carry
    lax.fori_loop(0, BPS, zrow, 0)
    pltpu.sync_copy(tmp_v, acc_sh.at[pl.ds(sid * BPS, BPS)])

    pltpu.sync_copy(ids_hbm.at[wid], ids_v)
    plsc.subcore_barrier()

    def chunk_body(j, carry):
      g = wid * CPW + j
      pltpu.sync_copy(rows_hbm.at[g], buf_v)
      pltpu.sync_copy(buf_v, acc_sh.at[ids_v.at[j]], add=True)
      return carry
    lax.fori_loop(0, CPW, chunk_body, 0)

    plsc.subcore_barrier()
    pltpu.sync_copy(acc_sh.at[pl.ds(sid * BPS, BPS)], tmp_v)
    pltpu.sync_copy(tmp_v, out_hbm.at[cid, pl.ds(sid * BPS, BPS)])

  return k(rows3, ids3)


def _combine(partials):
  def add_body(a_ref, b_ref, o_ref):
    o_ref[...] = a_ref[...] + b_ref[...]

  return pl.pallas_call(
      add_body,
      out_shape=jax.ShapeDtypeStruct((B, D), jnp.float32),
  )(partials[0], partials[1])


def kernel(outputs, atom_split):
  rows3 = outputs.reshape(NW * CPW, CHUNK, D)
  ids3 = atom_split.astype(jnp.int32).reshape(NW, CPW, CHUNK)
  partials = _sc_segment_sum(rows3, ids3)
  return _combine(partials)


# speedup vs baseline: 3.7245x; 1.1778x over previous
import functools

import jax
import jax.numpy as jnp
from jax import lax
from jax.experimental import pallas as pl
from jax.experimental.pallas import tpu as pltpu
from jax.experimental.pallas import tpu_sc as plsc

N = 320000
D = 128
B = 1024
NC = 2
NS = 16
NW = NC * NS
CHUNK = 125
CPW = N // NW // CHUNK
BPS = B // NS


def _sc_segment_sum(rows3, ids3):
  mesh = plsc.VectorSubcoreMesh(core_axis_name="c", subcore_axis_name="s")

  NBUF = 4
  SPB = CPW // NBUF

  @functools.partial(
      pl.kernel,
      mesh=mesh,
      out_type=jax.ShapeDtypeStruct((NC, B, D), jnp.float32),
      scratch_types=[
          pltpu.VMEM((CPW, CHUNK), jnp.int32),
          pltpu.VMEM((CHUNK, D), jnp.float32),
          pltpu.VMEM((CHUNK, D), jnp.float32),
          pltpu.VMEM((CHUNK, D), jnp.float32),
          pltpu.VMEM((CHUNK, D), jnp.float32),
          pltpu.VMEM((BPS, D), jnp.float32),
          pltpu.VMEM_SHARED((B, D), jnp.float32),
          pltpu.SemaphoreType.DMA,
          pltpu.SemaphoreType.DMA,
          pltpu.SemaphoreType.DMA,
          pltpu.SemaphoreType.DMA,
          pltpu.SemaphoreType.DMA,
          pltpu.SemaphoreType.DMA,
          pltpu.SemaphoreType.DMA,
          pltpu.SemaphoreType.DMA,
      ],
  )
  def k(rows_hbm, ids_hbm, out_hbm, ids_v, b0, b1, b2, b3, tmp_v, acc_sh,
        g0, g1, g2, g3, s0, s1, s2, s3):
    cid = lax.axis_index("c")
    sid = lax.axis_index("s")
    wid = cid * NS + sid
    bufs = (b0, b1, b2, b3)
    gsems = (g0, g1, g2, g3)
    ssems = (s0, s1, s2, s3)

    def zrow(r, carry):
      for c in range(D // 16):
        tmp_v[r, pl.ds(c * 16, 16)] = jnp.zeros((16,), jnp.float32)
      return carry
    lax.fori_loop(0, BPS, zrow, 0)
    pltpu.sync_copy(tmp_v, acc_sh.at[pl.ds(sid * BPS, BPS)])

    pltpu.sync_copy(ids_hbm.at[wid], ids_v)
    plsc.subcore_barrier()

    base = wid * CPW

    for b in range(NBUF):
      pltpu.make_async_copy(rows_hbm.at[base + b], bufs[b], gsems[b]).start()

    def outer(i, carry):
      j0 = i * NBUF
      scatters = []
      for b in range(NBUF):
        pltpu.make_async_copy(rows_hbm.at[base + j0 + b], bufs[b],
                              gsems[b]).wait()
        scatters.append(
            pltpu.async_copy(bufs[b], acc_sh.at[ids_v.at[j0 + b]], ssems[b],
                             add=True))
      for b in range(NBUF):
        scatters[b].wait()
        nxt = j0 + NBUF + b

        @pl.when(nxt < CPW)
        def _():
          pltpu.make_async_copy(rows_hbm.at[base + nxt], bufs[b],
                                gsems[b]).start()
      return carry

    lax.fori_loop(0, SPB, outer, 0)

    plsc.subcore_barrier()
    pltpu.sync_copy(acc_sh.at[pl.ds(sid * BPS, BPS)], tmp_v)
    pltpu.sync_copy(tmp_v, out_hbm.at[cid, pl.ds(sid * BPS, BPS)])

  return k(rows3, ids3)


def _combine(partials):
  def add_body(a_ref, b_ref, o_ref):
    o_ref[...] = a_ref[...] + b_ref[...]

  return pl.pallas_call(
      add_body,
      out_shape=jax.ShapeDtypeStruct((B, D), jnp.float32),
  )(partials[0], partials[1])


def kernel(outputs, atom_split):
  rows3 = outputs.reshape(NW * CPW, CHUNK, D)
  ids3 = atom_split.astype(jnp.int32).reshape(NW, CPW, CHUNK)
  partials = _sc_segment_sum(rows3, ids3)
  return _combine(partials)


# speedup vs baseline: 7.0913x; 1.9040x over previous
import functools

import jax
import jax.numpy as jnp
from jax import lax
from jax.experimental import pallas as pl
from jax.experimental.pallas import tpu as pltpu
from jax.experimental.pallas import tpu_sc as plsc

N = 320000
D = 128
B = 1024
NC = 2
NS = 16
NW = NC * NS
CHUNK = 80
CPW = N // NW // CHUNK
BPS = B // NS


def _sc_segment_sum(rows3, ids3):
  mesh = plsc.VectorSubcoreMesh(core_axis_name="c", subcore_axis_name="s")

  NBUF = 5
  SPB = CPW // NBUF

  @functools.partial(
      pl.kernel,
      mesh=mesh,
      out_type=jax.ShapeDtypeStruct((NC, B, D), jnp.float32),
      scratch_types=[
          pltpu.VMEM((CPW, CHUNK), jnp.int32),
          pltpu.VMEM((CHUNK, D), jnp.float32),
          pltpu.VMEM((CHUNK, D), jnp.float32),
          pltpu.VMEM((CHUNK, D), jnp.float32),
          pltpu.VMEM((CHUNK, D), jnp.float32),
          pltpu.VMEM((CHUNK, D), jnp.float32),
          pltpu.VMEM((BPS, D), jnp.float32),
          pltpu.VMEM_SHARED((B, D), jnp.float32),
          pltpu.SemaphoreType.DMA,
          pltpu.SemaphoreType.DMA,
          pltpu.SemaphoreType.DMA,
          pltpu.SemaphoreType.DMA,
          pltpu.SemaphoreType.DMA,
          pltpu.SemaphoreType.DMA,
          pltpu.SemaphoreType.DMA,
          pltpu.SemaphoreType.DMA,
          pltpu.SemaphoreType.DMA,
          pltpu.SemaphoreType.DMA,
      ],
  )
  def k(rows_hbm, ids_hbm, out_hbm, ids_v, b0, b1, b2, b3, b4, tmp_v, acc_sh,
        g0, g1, g2, g3, g4, s0, s1, s2, s3, s4):
    cid = lax.axis_index("c")
    sid = lax.axis_index("s")
    wid = cid * NS + sid
    bufs = (b0, b1, b2, b3, b4)
    gsems = (g0, g1, g2, g3, g4)
    ssems = (s0, s1, s2, s3, s4)

    def zrow(r, carry):
      for c in range(D // 16):
        tmp_v[r, pl.ds(c * 16, 16)] = jnp.zeros((16,), jnp.float32)
      return carry
    lax.fori_loop(0, BPS, zrow, 0)
    pltpu.sync_copy(tmp_v, acc_sh.at[pl.ds(sid * BPS, BPS)])

    pltpu.sync_copy(ids_hbm.at[wid], ids_v)
    plsc.subcore_barrier()

    base = wid * (CPW * CHUNK)

    for b in range(NBUF):
      pltpu.make_async_copy(rows_hbm.at[pl.ds(base + b * CHUNK, CHUNK)],
                            bufs[b], gsems[b]).start()

    def outer(i, carry):
      j0 = i * NBUF
      scatters = []
      for b in range(NBUF):
        pltpu.make_async_copy(
            rows_hbm.at[pl.ds(base + (j0 + b) * CHUNK, CHUNK)], bufs[b],
            gsems[b]).wait()
        scatters.append(
            pltpu.async_copy(bufs[b], acc_sh.at[ids_v.at[j0 + b]], ssems[b],
                             add=True))
      for b in range(NBUF):
        scatters[b].wait()
        nxt = j0 + NBUF + b

        @pl.when(nxt < CPW)
        def _():
          pltpu.make_async_copy(
              rows_hbm.at[pl.ds(base + nxt * CHUNK, CHUNK)], bufs[b],
              gsems[b]).start()
      return carry

    lax.fori_loop(0, SPB, outer, 0)

    plsc.subcore_barrier()
    pltpu.sync_copy(acc_sh.at[pl.ds(sid * BPS, BPS)], tmp_v)
    pltpu.sync_copy(tmp_v, out_hbm.at[cid, pl.ds(sid * BPS, BPS)])

  return k(rows3, ids3)


def _combine(partials):
  def add_body(a_ref, b_ref, o_ref):
    o_ref[...] = a_ref[...] + b_ref[...]

  return pl.pallas_call(
      add_body,
      out_shape=jax.ShapeDtypeStruct((B, D), jnp.float32),
  )(partials[0], partials[1])


def kernel(outputs, atom_split):
  ids3 = atom_split.astype(jnp.int32).reshape(NW, CPW, CHUNK)
  partials = _sc_segment_sum(outputs, ids3)
  return _combine(partials)
